# baseline (speedup 1.0000x reference)
import functools

import jax
import jax.numpy as jnp
from jax import lax
from jax.experimental import pallas as pl
from jax.experimental.pallas import tpu as pltpu
from jax.experimental.pallas import tpu_sc as plsc

_NC, _NS, _L = 2, 16, 16
_NW = _NC * _NS
_CH = 16
_UNROLL = 8


def _make_sc_kernel(B, S, D):
    assert B == 4
    spw = S // _NW
    n_chunks = spw // _CH
    vregs = _CH * D // _L

    mesh = plsc.VectorSubcoreMesh(
        core_axis_name="c", subcore_axis_name="s",
        num_cores=_NC, num_subcores=_NS,
    )

    @functools.partial(
        pl.kernel,
        out_type=jax.ShapeDtypeStruct((B, S, D), jnp.float32),
        mesh=mesh,
        scratch_types=[
            pltpu.VMEM((4 * _CH, D), jnp.float32),
            pltpu.VMEM((2 * _CH, D), jnp.float32),
            [pltpu.SemaphoreType.DMA] * 4,
            [pltpu.SemaphoreType.DMA] * 4,
            pltpu.SemaphoreType.DMA,
        ],
        compiler_params=pltpu.CompilerParams(use_tc_tiling_on_sc=True),
    )
    def sc_add(x_hbm, emb_hbm, out_hbm, xring, embbuf, in_sems, out_sems,
               emb_sem):
        wid = lax.axis_index("s") * _NC + lax.axis_index("c")
        row_base = wid * spw

        def issue_in(g, j, slot):
            pltpu.async_copy(
                x_hbm.at[j, pl.ds(row_base + g * _CH, _CH)],
                xring.at[pl.ds(slot * _CH, _CH)],
                in_sems[slot],
            )

        def issue_out_half(g, j, slot, h):
            hh = _CH // 2
            pltpu.async_copy(
                xring.at[pl.ds(slot * _CH + h * hh, hh)],
                out_hbm.at[j, pl.ds(row_base + g * _CH + h * hh, hh)],
                out_sems[slot],
            )

        def wait_in(slot):
            pltpu.make_async_copy(
                x_hbm.at[0, pl.ds(0, _CH)],
                xring.at[pl.ds(slot * _CH, _CH)],
                in_sems[slot],
            ).wait()

        def wait_out(slot):
            pltpu.make_async_copy(
                xring.at[pl.ds(slot * _CH, _CH)],
                out_hbm.at[0, pl.ds(0, _CH)],
                out_sems[slot],
            ).wait()

        def issue_emb(g):
            pltpu.async_copy(
                emb_hbm.at[pl.ds(row_base + g * _CH, _CH)],
                embbuf.at[pl.ds((g % 2) * _CH, _CH)],
                emb_sem,
            )

        def wait_emb():
            pltpu.make_async_copy(
                emb_hbm.at[pl.ds(0, _CH)],
                embbuf.at[pl.ds(0, _CH)],
                emb_sem,
            ).wait()

        issue_emb(0)
        issue_in(0, 0, 0)
        issue_in(0, 1, 1)

        def group(g, _):
            wait_emb()
            ebase = (g % 2) * _CH

            @pl.when(g < n_chunks - 1)
            def _():
                issue_emb(g + 1)

            for j in range(4):
                jp2 = (j + 2) % 4
                if j < 2:
                    @pl.when(g > 0)
                    def _():
                        wait_out(jp2)
                    issue_in(g, j + 2, jp2)
                else:
                    wait_out(jp2)

                    @pl.when(g < n_chunks - 1)
                    def _():
                        issue_in(g + 1, j - 2, jp2)

                wait_in(j)

                def make_row_loop(half):
                    def row_loop(r, _):
                        xrow = j * _CH + half * (_CH // 2) + r
                        erow = ebase + half * (_CH // 2) + r

                        @plsc.parallel_loop(0, D // _L, unroll=_UNROLL)
                        def col_loop(c):
                            plsc.addupdate(
                                xring.at[xrow, pl.ds(c * _L, _L)],
                                embbuf[erow, pl.ds(c * _L, _L)],
                            )

                        return 0

                    return row_loop

                lax.fori_loop(0, _CH // 2, make_row_loop(0), 0)
                issue_out_half(g, j, j, 0)
                lax.fori_loop(0, _CH // 2, make_row_loop(1), 0)
                issue_out_half(g, j, j, 1)
            return 0

        lax.fori_loop(0, n_chunks, group, 0)
        wait_out(2)
        wait_out(3)

    return sc_add


def kernel(x, emb_weight):
    B, S, D = x.shape
    sc_add = _make_sc_kernel(B, S, D)
    return sc_add(x, emb_weight)


# speedup vs baseline: 1.0014x; 1.0014x over previous
import functools

import jax
import jax.numpy as jnp
from jax import lax
from jax.experimental import pallas as pl
from jax.experimental.pallas import tpu as pltpu
from jax.experimental.pallas import tpu_sc as plsc

_NC, _NS, _L = 2, 16, 16
_NW = _NC * _NS
_CH = 16
_UNROLL = 8


def _make_sc_kernel(B, S, D):
    assert B == 4 and S % (_NW * _CH) == 0 and D % (_L * _UNROLL) == 0
    spw = S // _NW
    n_chunks = spw // _CH

    mesh = plsc.VectorSubcoreMesh(
        core_axis_name="c", subcore_axis_name="s",
        num_cores=_NC, num_subcores=_NS,
    )

    @functools.partial(
        pl.kernel,
        out_type=jax.ShapeDtypeStruct((B, S, D), jnp.float32),
        mesh=mesh,
        scratch_types=[
            pltpu.VMEM((4 * _CH, D), jnp.float32),
            pltpu.VMEM((2 * _CH, D), jnp.float32),
            [pltpu.SemaphoreType.DMA] * 4,
            [pltpu.SemaphoreType.DMA] * 4,
            pltpu.SemaphoreType.DMA,
        ],
        compiler_params=pltpu.CompilerParams(use_tc_tiling_on_sc=True),
    )
    def sc_add(x_hbm, emb_hbm, out_hbm, xring, embbuf, in_sems, out_sems,
               emb_sem):
        wid = lax.axis_index("s") * _NC + lax.axis_index("c")
        row_base = wid * spw

        def issue_in(g, j, slot):
            pltpu.async_copy(
                x_hbm.at[j, pl.ds(row_base + g * _CH, _CH)],
                xring.at[pl.ds(slot * _CH, _CH)],
                in_sems[slot],
            )

        def issue_out(g, j, slot):
            pltpu.async_copy(
                xring.at[pl.ds(slot * _CH, _CH)],
                out_hbm.at[j, pl.ds(row_base + g * _CH, _CH)],
                out_sems[slot],
            )

        def wait_in(slot):
            pltpu.make_async_copy(
                x_hbm.at[0, pl.ds(0, _CH)],
                xring.at[pl.ds(slot * _CH, _CH)],
                in_sems[slot],
            ).wait()

        def wait_out(slot):
            pltpu.make_async_copy(
                xring.at[pl.ds(slot * _CH, _CH)],
                out_hbm.at[0, pl.ds(0, _CH)],
                out_sems[slot],
            ).wait()

        def issue_emb(g):
            pltpu.async_copy(
                emb_hbm.at[pl.ds(row_base + g * _CH, _CH)],
                embbuf.at[pl.ds((g % 2) * _CH, _CH)],
                emb_sem,
            )

        def wait_emb():
            pltpu.make_async_copy(
                emb_hbm.at[pl.ds(0, _CH)],
                embbuf.at[pl.ds(0, _CH)],
                emb_sem,
            ).wait()

        issue_emb(0)
        issue_in(0, 0, 0)
        issue_in(0, 1, 1)

        def group(g, _):
            wait_emb()
            ebase = (g % 2) * _CH

            @pl.when(g < n_chunks - 1)
            def _():
                issue_emb(g + 1)

            for j in range(4):
                jp2 = (j + 2) % 4
                if j < 2:
                    @pl.when(g > 0)
                    def _():
                        wait_out(jp2)
                    issue_in(g, j + 2, jp2)
                else:
                    wait_out(jp2)

                    @pl.when(g < n_chunks - 1)
                    def _():
                        issue_in(g + 1, j - 2, jp2)

                wait_in(j)

                def row_loop(r, _):
                    xrow = j * _CH + r
                    erow = ebase + r

                    @plsc.parallel_loop(0, D // _L, unroll=_UNROLL)
                    def col_loop(c):
                        plsc.addupdate(
                            xring.at[xrow, pl.ds(c * _L, _L)],
                            embbuf[erow, pl.ds(c * _L, _L)],
                        )

                    return 0

                lax.fori_loop(0, _CH, row_loop, 0)
                issue_out(g, j, j)
            return 0

        lax.fori_loop(0, n_chunks, group, 0)
        wait_out(2)
        wait_out(3)

    return sc_add


def kernel(x, emb_weight):
    B, S, D = x.shape
    sc_add = _make_sc_kernel(B, S, D)
    return sc_add(x, emb_weight)
